# Initial kernel scaffold; baseline (speedup 1.0000x reference)
import jax
import jax.numpy as jnp
from jax.experimental import pallas as pl


def kernel(outputs, emb, doc_ids):
    raise NotImplementedError("write your pallas kernel here")


# speedup vs baseline: 198.1392x; 198.1392x over previous
import numpy as np
import jax
import jax.numpy as jnp
from jax import lax
from jax.experimental import pallas as pl
from jax.experimental.pallas import tpu as pltpu


_KEYWORDS_LIST = [["grp%d" % (i % 8), "cat%d" % (i // 8)] for i in range(32)]
_BATCH, _SEQ, _HIDDEN = 64, 128, 1024


class _Node:
    def __init__(self, keyword, idf, covered_document_idx, is_end=False):
        self.keyword = keyword
        self.idf = idf
        self.covered_document_idx = covered_document_idx
        self.children = None
        self.is_end = is_end


def _normalize_keywords(keywords):
    return tuple(sorted(set(keywords)))


def _find_most_common_keyword(documents):
    frequency = {}
    for doc in documents:
        if doc is not None:
            for kw in doc:
                if kw not in frequency:
                    frequency[kw] = 0
                frequency[kw] += 1
    items = sorted(list(frequency.items()), key=lambda x: x[1], reverse=True)
    if len(items) > 0:
        return items[0][0], float(np.log(len(documents) / (1 + items[0][1])))
    return None, 0


def _cover_documents(documents):
    covered, not_covered = [], []
    cov_idx = set()
    kw, idf = _find_most_common_keyword(documents)
    if kw is None:
        raise ValueError('No common keyword found')
    for idx, doc in enumerate(documents):
        if doc and kw in doc:
            cov_idx.add(idx)
            covered.append(tuple(k for k in doc if k != kw))
            not_covered.append(None)
        else:
            covered.append(None)
            not_covered.append(doc)
    return kw, idf, covered, cov_idx, not_covered


def _make_tree(root, documents):
    nodes, children, children_cov = [], [], []
    not_covered = documents
    all_cov = set(i for i, d in enumerate(documents) if d is not None and not d)
    while any(not_covered):
        kw, idf, cov_docs, cov_idx, not_covered = _cover_documents(not_covered)
        children.append(_Node(kw, idf, cov_idx))
        children_cov.append(cov_docs)
    nodes += children
    for child, cdocs in zip(children, children_cov):
        _, added = _make_tree(child, cdocs)
        nodes += added
    if all_cov:
        end = [_Node(None, 0, all_cov, is_end=True)]
        children = end + children
        nodes = end + nodes
    root.children = children if len(children) > 0 else None
    return root, nodes


_DOCS = [_normalize_keywords(k) for k in _KEYWORDS_LIST]
_root = _Node(None, 0, set(range(len(_DOCS))))
_ROOT, _nodes = _make_tree(_root, _DOCS)
_NODES = [_ROOT] + _nodes
_NN = len(_NODES)


def _index_of(lst, obj):
    for i, x in enumerate(lst):
        if x is obj:
            return i
    raise ValueError('node not found')


def _doc_path(d):
    node = _ROOT
    path = []
    while node.children is not None:
        nxt = [c for c in node.children if d in c.covered_document_idx][0]
        off = _index_of(_NODES, node.children[0])
        n = len(node.children)
        tgt = _index_of(_NODES, nxt) - off
        path.append((off, n, tgt))
        node = nxt
    return path


_PATHS = [_doc_path(d) for d in range(len(_DOCS))]
_MAX_STEPS = max(len(p) for p in _PATHS)
_NDOC = len(_DOCS)

_NP = 128
assert _NN <= _NP

_W = np.zeros((_MAX_STEPS * _NDOC, _NP), dtype=np.float32)
_T = np.zeros((_MAX_STEPS * _NDOC, _NP), dtype=np.float32)
_SINV = np.zeros((_NDOC, 1), dtype=np.float32)
for _d, _p in enumerate(_PATHS):
    _SINV[_d, 0] = 1.0 / len(_p)
    for _s in range(_MAX_STEPS):
        r = _s * _NDOC + _d
        if _s < len(_p):
            _o, _n, _t = _p[_s]
            idx = np.clip(_o + np.arange(_n), 0, _NN - 1)
            _W[r, idx] = 1.0
            _T[r, np.clip(_o + _t, 0, _NN - 1)] = 1.0
        else:
            _W[r, 0] = 1.0
            _T[r, 0] = 1.0

_W_J = jnp.asarray(_W)
_T_J = jnp.asarray(_T)
_SINV_J = jnp.asarray(_SINV)


def _tc_body(h_ref, e_ref, d_ref, w_ref, t_ref, sinv_ref, out_ref):
    h = h_ref[...]
    e = e_ref[...]
    g = lax.dot_general(h, e, (((1,), (1,)), ((), ())),
                        preferred_element_type=jnp.float32)
    d = d_ref[...]
    oh = (d == lax.broadcasted_iota(jnp.int32, (_BATCH, _NDOC), 1)
          ).astype(jnp.float32)
    acc = jnp.zeros((_BATCH, 1), dtype=jnp.float32)
    for s in range(_MAX_STEPS):
        w = jnp.dot(oh, w_ref[s * _NDOC:(s + 1) * _NDOC, :],
                    preferred_element_type=jnp.float32)
        t = jnp.dot(oh, t_ref[s * _NDOC:(s + 1) * _NDOC, :],
                    preferred_element_type=jnp.float32)
        masked = jnp.where(w > 0.5, g, -1e30)
        mx = jnp.max(masked, axis=1, keepdims=True)
        ssum = jnp.sum(jnp.where(w > 0.5, jnp.exp(masked - mx), 0.0),
                       axis=1, keepdims=True)
        pick = jnp.sum(g * t, axis=1, keepdims=True)
        acc = acc + mx + jnp.log(ssum) - pick
    sinv = jnp.dot(oh, sinv_ref[...],
                   preferred_element_type=jnp.float32)
    out_ref[...] = jnp.sum(acc * sinv, keepdims=True) * (1.0 / _BATCH)


def kernel(outputs, emb, doc_ids):
    hidden = outputs[:, 0, :]
    embp = jnp.zeros((_NP, _HIDDEN), dtype=jnp.float32).at[:_NN].set(emb)
    d2 = doc_ids.astype(jnp.int32).reshape(_BATCH, 1)
    out = pl.pallas_call(
        _tc_body,
        out_shape=jax.ShapeDtypeStruct((1, 1), jnp.float32),
        in_specs=[
            pl.BlockSpec((_BATCH, _HIDDEN), lambda: (0, 0)),
            pl.BlockSpec((_NP, _HIDDEN), lambda: (0, 0)),
            pl.BlockSpec((_BATCH, 1), lambda: (0, 0)),
            pl.BlockSpec((_MAX_STEPS * _NDOC, _NP), lambda: (0, 0)),
            pl.BlockSpec((_MAX_STEPS * _NDOC, _NP), lambda: (0, 0)),
            pl.BlockSpec((_NDOC, 1), lambda: (0, 0)),
        ],
        out_specs=pl.BlockSpec((1, 1), lambda: (0, 0)),
    )(hidden, embp, d2, _W_J, _T_J, _SINV_J)
    return out[0, 0]
